# baseline (speedup 1.0000x reference)
import jax, jax.numpy as jnp
import numpy as np
from jax.experimental import pallas as pl

N, E, G, H, SPEC, NCONV, CUTOFF = 10000, 320000, 100, 128, 1000, 4, 10.0


def _mul_kernel(a_ref, b_ref, o_ref):
    o_ref[...] = a_ref[...] * b_ref[...]


def _pl_mul(a, b):
    blk = (2000, a.shape[1])
    grid = (a.shape[0] // blk[0],)
    spec = pl.BlockSpec(blk, lambda i: (i, 0))
    return pl.pallas_call(
        _mul_kernel,
        grid=grid,
        in_specs=[spec, spec],
        out_specs=spec,
        out_shape=jax.ShapeDtypeStruct(a.shape, a.dtype),
    )(a, b)


def kernel(atom_type, edge_index, edge_type, batch_idx, pos_perturbed, time_step, spectrum, params):
    P = params
    pos = pos_perturbed
    ssp = lambda x: jax.nn.softplus(x) - jnp.log(2.0)
    spec = jax.nn.relu(spectrum @ P["Ws1"] + P["bs1"]) @ P["Ws2"] + P["bs2"]
    half = H // 2
    freq = jnp.exp(jnp.arange(half, dtype=jnp.float32) * (-np.log(10000.0) / (half - 1)))
    te = time_step.astype(jnp.float32)[:, None] * freq[None, :]
    te = jnp.concatenate([jnp.sin(te), jnp.cos(te)], axis=-1)
    tc = jax.nn.gelu(te @ P["Wt1"] + P["bt1"]) @ P["Wt2"] + P["bt2"]
    comb = jnp.concatenate([spec, tc], axis=1)
    cond = jax.nn.gelu(comb @ P["Wc1"] + P["bc1"]) @ P["Wc2"] + P["bc2"]
    node_cond = cond[batch_idx]
    src = edge_index[0]
    dst = edge_index[1]
    d = pos[dst] - pos[src]
    edge_length = jnp.sqrt(jnp.sum(d * d, axis=-1, keepdims=True) + 1e-12)
    mask = edge_type > 0
    mf = mask.astype(jnp.float32)[:, None]
    eg = (jax.nn.relu(edge_length @ P["Weg1"] + P["beg1"]) @ P["Weg2"] + P["beg2"]) * P["EmbEg"][edge_type]
    el = (jax.nn.relu(edge_length @ P["Wel1"] + P["bel1"]) @ P["Wel2"] + P["bel2"]) * P["EmbEl"][edge_type]
    C = 0.5 * (jnp.cos(edge_length * jnp.pi / CUTOFF) + 1.0) * (edge_length <= CUTOFF).astype(jnp.float32)
    h = P["EmbG"][atom_type]
    for i in range(NCONV):
        W = ssp(eg @ P["Wf1"][i] + P["bf1"][i]) @ P["Wf2"][i] + P["bf2"][i]
        x = h @ P["Wl1"][i]
        msg = _pl_mul(x[src], W * C)
        agg = jax.ops.segment_sum(msg, dst, num_segments=N)
        h = h + (ssp(agg @ P["Wl2"][i] + P["bl2"][i]) @ P["Wl3"][i] + P["bl3"][i])
    node_g = h + node_cond
    hp_g = jnp.concatenate([node_g[src] * node_g[dst], eg], axis=-1)
    eig = jax.nn.relu(jax.nn.relu(hp_g @ P["Wg1"] + P["bg1"]) @ P["Wg2"] + P["bg2"]) @ P["Wg3"] + P["bg3"]
    hl = P["EmbL"][atom_type]
    for i in range(NCONV):
        msg = jax.nn.relu(hl[src] + el) * mf
        agg = jax.ops.segment_sum(msg, dst, num_segments=N)
        hl = jax.nn.relu((hl + agg) @ P["Wgin1"][i] + P["bgin1"][i]) @ P["Wgin2"][i] + P["bgin2"][i]
    node_l = hl + node_cond
    hp_l = jnp.concatenate([node_l[src] * node_l[dst], el], axis=-1)
    eil = (jax.nn.relu(jax.nn.relu(hp_l @ P["WL1"] + P["bL1"]) @ P["WL2"] + P["bL2"]) @ P["WL3"] + P["bL3"]) * mf
    return (eig, eil, edge_index, edge_type, edge_length, mask)


# speedup vs baseline: 1.2044x; 1.2044x over previous
import functools
import jax
import jax.numpy as jnp
import numpy as np
from jax import lax
from jax.experimental import pallas as pl
from jax.experimental.pallas import tpu as pltpu
from jax.experimental.pallas import tpu_sc as plsc

N, E, G, H, SPEC, NCONV, CUTOFF = 10000, 320000, 100, 128, 1000, 4, 10.0

NW = 32
CH = 128
EP = 323584
EW = EP // NW
NCHUNK = EW // CH
NACC = 10240
RPT = NACC // 16

EBLK = 1024
NBLK = 2000

_LN2 = float(np.log(2.0))

def _ssp(x):
    return jnp.maximum(x, 0.0) + jnp.log1p(jnp.exp(-jnp.abs(x))) - _LN2


def _cond_body(ts_ref, freq_ref, spec_ref, Ws1, bs1, Ws2, bs2, Wt1, bt1, Wt2, bt2,
               Wc1a, Wc1b, bc1, Wc2, bc2, out_ref):
    ph = ts_ref[...] * freq_ref[...]
    te = jnp.concatenate([jnp.sin(ph), jnp.cos(ph)], axis=1)
    tcv = jax.nn.gelu(te @ Wt1[...] + bt1[...]) @ Wt2[...] + bt2[...]
    sp = jax.nn.relu(spec_ref[...] @ Ws1[...] + bs1[...]) @ Ws2[...] + bs2[...]
    pre = sp @ Wc1a[...] + tcv @ Wc1b[...] + bc1[...]
    out_ref[...] = jax.nn.gelu(pre) @ Wc2[...] + bc2[...]


def _cond(ts, spectrum, P):
    Wc1a = P["Wc1"][:H]
    Wc1b = P["Wc1"][H:]
    half = H // 2
    freq = jnp.exp(jnp.arange(half, dtype=jnp.float32)
                   * (-np.log(10000.0) / (half - 1)))[None]
    return pl.pallas_call(
        _cond_body,
        out_shape=jax.ShapeDtypeStruct((G, H), jnp.float32),
    )(ts, freq, spectrum, P["Ws1"], P["bs1"][None], P["Ws2"], P["bs2"][None],
      P["Wt1"], P["bt1"][None], P["Wt2"], P["bt2"][None],
      Wc1a, Wc1b, P["bc1"][None], P["Wc2"], P["bc2"][None])


def _init_body(atom_ref, EmbG, EmbL, Wl1_0, h0_ref, hl0_ref, x0_ref):
    oh = (atom_ref[...] == lax.broadcasted_iota(jnp.int32, (1, 128), 1)
          ).astype(jnp.float32)
    h0 = jnp.dot(oh, EmbG[...], precision=lax.Precision.HIGHEST)
    h0_ref[...] = h0
    hl0_ref[...] = jnp.dot(oh, EmbL[...], precision=lax.Precision.HIGHEST)
    x0_ref[...] = h0 @ Wl1_0[...]


def _node_init(atom2, EmbG_p, EmbL_p, Wl1_0):
    grid = (N // NBLK,)
    bspec = pl.BlockSpec((NBLK, H), lambda i: (i, 0))
    wspec = pl.BlockSpec((128, H), lambda i: (0, 0))
    return pl.pallas_call(
        _init_body,
        grid=grid,
        in_specs=[pl.BlockSpec((NBLK, 1), lambda i: (i, 0)), wspec, wspec, wspec],
        out_specs=[bspec, bspec, bspec],
        out_shape=[jax.ShapeDtypeStruct((N, H), jnp.float32)] * 3,
    )(atom2, EmbG_p, EmbL_p, Wl1_0)


def _encode_body(ln_ref, et_ref, dst_ref,
                 Weg1, beg1, Weg2, beg2, EmbEg,
                 Wel1, bel1, Wel2, bel2, EmbEl,
                 Wf1, bf1, Wf2, bf2,
                 eg_ref, el_ref, wc0_ref, wc1_ref, wc2_ref, wc3_ref,
                 mf_ref, sdst_ref, gdst_ref):
    ln = ln_ref[...]
    et = et_ref[...]
    oh = (et == lax.broadcasted_iota(jnp.int32, (1, 128), 1)).astype(jnp.float32)
    emb_g = jnp.dot(oh, EmbEg[...], precision=lax.Precision.HIGHEST)
    emb_l = jnp.dot(oh, EmbEl[...], precision=lax.Precision.HIGHEST)
    eg = (jax.nn.relu(ln * Weg1[...] + beg1[...]) @ Weg2[...] + beg2[...]) * emb_g
    el = (jax.nn.relu(ln * Wel1[...] + bel1[...]) @ Wel2[...] + bel2[...]) * emb_l
    eg_ref[...] = eg
    el_ref[...] = el
    C = 0.5 * (jnp.cos(ln * (np.pi / CUTOFF)) + 1.0) * (ln <= CUTOFF).astype(jnp.float32)
    wrefs = (wc0_ref, wc1_ref, wc2_ref, wc3_ref)
    for i in range(NCONV):
        w = _ssp(eg @ Wf1[i] + bf1[i][None]) @ Wf2[i] + bf2[i][None]
        wrefs[i][...] = w * C
    pid = pl.program_id(0)
    row = pid * EBLK + lax.broadcasted_iota(jnp.int32, (EBLK, 1), 0)
    valid = row < E
    bonded = jnp.logical_and(valid, et > 0)
    mf_ref[...] = jnp.logical_and(valid, et > 0).astype(jnp.float32)
    dst = dst_ref[...]
    sdst_ref[...] = jnp.where(valid, dst, N)
    gdst_ref[...] = jnp.where(bonded, dst, N)


def _edge_encode(lnp, et2, dst2, P):
    grid = (EP // EBLK,)
    col1 = pl.BlockSpec((EBLK, 1), lambda i: (i, 0))
    colH = pl.BlockSpec((EBLK, H), lambda i: (i, 0))
    w1 = pl.BlockSpec((1, H), lambda i: (0, 0))
    wHH = pl.BlockSpec((H, H), lambda i: (0, 0))
    w128 = pl.BlockSpec((128, H), lambda i: (0, 0))
    wc = pl.BlockSpec((NCONV, H, H), lambda i: (0, 0, 0))
    wcb = pl.BlockSpec((NCONV, H), lambda i: (0, 0))
    fH = jax.ShapeDtypeStruct((EP, H), jnp.float32)
    f1 = jax.ShapeDtypeStruct((EP, 1), jnp.float32)
    i1 = jax.ShapeDtypeStruct((EP, 1), jnp.int32)
    EmbEg_p = jnp.zeros((128, H), jnp.float32).at[:100].set(P["EmbEg"])
    EmbEl_p = jnp.zeros((128, H), jnp.float32).at[:100].set(P["EmbEl"])
    return pl.pallas_call(
        _encode_body,
        grid=grid,
        in_specs=[col1, col1, col1,
                  w1, w1, wHH, w1, w128,
                  w1, w1, wHH, w1, w128,
                  wc, wcb, wc, wcb],
        out_specs=[colH, colH, colH, colH, colH, colH, col1, col1, col1],
        out_shape=[fH, fH, fH, fH, fH, fH, f1, i1, i1],
    )(lnp, et2, dst2,
      P["Weg1"], P["beg1"][None], P["Weg2"], P["beg2"][None], EmbEg_p,
      P["Wel1"], P["bel1"][None], P["Wel2"], P["bel2"][None], EmbEl_p,
      P["Wf1"], P["bf1"], P["Wf2"], P["bf2"])


def _schnet_upd_body(last, p0_ref, p1_ref, h_ref, Wl2, bl2, Wl3, bl3, W_extra,
                     batch_ref, *out_refs):
    agg = p0_ref[...] + p1_ref[...]
    t = _ssp(agg @ Wl2[...] + bl2[...]) @ Wl3[...] + bl3[...]
    hn = h_ref[...] + t
    if last:
        oh = (batch_ref[...] == lax.broadcasted_iota(jnp.int32, (1, 128), 1)
              ).astype(jnp.float32)
        out_refs[0][...] = hn + jnp.dot(oh, W_extra[...], precision=lax.Precision.HIGHEST)
    else:
        out_refs[0][...] = hn
        out_refs[1][...] = hn @ W_extra[...]


def _schnet_update(p0, p1, h, P, i, batch2, cond_p):
    last = i == NCONV - 1
    grid = (N // NBLK,)
    bspec = pl.BlockSpec((NBLK, H), lambda k: (k, 0))
    wspec = pl.BlockSpec((H, H), lambda k: (0, 0))
    b1 = pl.BlockSpec((1, H), lambda k: (0, 0))
    col1 = pl.BlockSpec((NBLK, 1), lambda k: (k, 0))
    W_extra = cond_p if last else P["Wl1"][i + 1]
    n_out = 1 if last else 2
    return pl.pallas_call(
        functools.partial(_schnet_upd_body, last),
        grid=grid,
        in_specs=[bspec, bspec, bspec, wspec, b1, wspec, b1,
                  pl.BlockSpec((128, H), lambda k: (0, 0)) if last else wspec,
                  col1],
        out_specs=[bspec] * n_out,
        out_shape=[jax.ShapeDtypeStruct((N, H), jnp.float32)] * n_out,
    )(p0, p1, h, P["Wl2"][i], P["bl2"][i][None], P["Wl3"][i], P["bl3"][i][None],
      W_extra, batch2)


def _gin_upd_body(last, p0_ref, p1_ref, hl_ref, W1, b1, W2, b2, cond_p,
                  batch_ref, *out_refs):
    agg = p0_ref[...] + p1_ref[...]
    hn = jax.nn.relu((hl_ref[...] + agg) @ W1[...] + b1[...]) @ W2[...] + b2[...]
    if last:
        oh = (batch_ref[...] == lax.broadcasted_iota(jnp.int32, (1, 128), 1)
              ).astype(jnp.float32)
        hn = hn + jnp.dot(oh, cond_p[...], precision=lax.Precision.HIGHEST)
    out_refs[0][...] = hn


def _gin_update(p0, p1, hl, P, i, batch2, cond_p):
    last = i == NCONV - 1
    grid = (N // NBLK,)
    bspec = pl.BlockSpec((NBLK, H), lambda k: (k, 0))
    wspec = pl.BlockSpec((H, H), lambda k: (0, 0))
    b1 = pl.BlockSpec((1, H), lambda k: (0, 0))
    col1 = pl.BlockSpec((NBLK, 1), lambda k: (k, 0))
    return pl.pallas_call(
        functools.partial(_gin_upd_body, last),
        grid=grid,
        in_specs=[bspec, bspec, bspec, wspec, b1, wspec, b1,
                  pl.BlockSpec((128, H), lambda k: (0, 0)), col1],
        out_specs=[bspec],
        out_shape=[jax.ShapeDtypeStruct((N, H), jnp.float32)],
    )(p0, p1, hl, P["Wgin1"][i], P["bgin1"][i][None], P["Wgin2"][i],
      P["bgin2"][i][None], cond_p, batch2)


def _final_body(pg_ref, eg_ref, plr_ref, el_ref, mf_ref,
                Wg1a, Wg1b, bg1, Wg2, bg2, Wg3, bg3,
                WL1a, WL1b, bL1, WL2, bL2, WL3, bL3,
                eig_ref, eil_ref):
    t = jax.nn.relu(pg_ref[...] @ Wg1a[...] + eg_ref[...] @ Wg1b[...] + bg1[...])
    t = jax.nn.relu(t @ Wg2[...] + bg2[...])
    eig_ref[...] = t @ Wg3[...] + bg3[...]
    u = jax.nn.relu(plr_ref[...] @ WL1a[...] + el_ref[...] @ WL1b[...] + bL1[...])
    u = jax.nn.relu(u @ WL2[...] + bL2[...])
    eil_ref[...] = (u @ WL3[...] + bL3[...]) * mf_ref[...]


def _final_edges(pair_g, eg, pair_l, el, mf, P):
    grid = (EP // EBLK,)
    colH = pl.BlockSpec((EBLK, H), lambda i: (i, 0))
    col1 = pl.BlockSpec((EBLK, 1), lambda i: (i, 0))
    wHH = pl.BlockSpec((H, H), lambda i: (0, 0))
    wH64 = pl.BlockSpec((H, 64), lambda i: (0, 0))
    w641 = pl.BlockSpec((64, 1), lambda i: (0, 0))
    b1H = pl.BlockSpec((1, H), lambda i: (0, 0))
    b164 = pl.BlockSpec((1, 64), lambda i: (0, 0))
    b11 = pl.BlockSpec((1, 1), lambda i: (0, 0))
    f1 = jax.ShapeDtypeStruct((EP, 1), jnp.float32)
    return pl.pallas_call(
        _final_body,
        grid=grid,
        in_specs=[colH, colH, colH, colH, col1,
                  wHH, wHH, b1H, wH64, b164, w641, b11,
                  wHH, wHH, b1H, wH64, b164, w641, b11],
        out_specs=[col1, col1],
        out_shape=[f1, f1],
    )(pair_g, eg, pair_l, el, mf,
      P["Wg1"][:H], P["Wg1"][H:], P["bg1"][None], P["Wg2"], P["bg2"][None],
      P["Wg3"], P["bg3"][None],
      P["WL1"][:H], P["WL1"][H:], P["bL1"][None], P["WL2"], P["bL2"][None],
      P["WL3"], P["bL3"][None])


def _sc_mesh():
    return plsc.VectorSubcoreMesh(core_axis_name="c", subcore_axis_name="s")


def _conv_body(is_gin, table_hbm, feat_hbm, src3_hbm, dst3_hbm, fidx3_hbm,
               p_hbm, acc, rows, feat_v, sbuf, dbuf, fbuf, sem):
    cid = lax.axis_index("c")
    sid = lax.axis_index("s")
    wid = sid * 2 + cid

    zero16 = jnp.zeros((16,), jnp.float32)

    @pl.loop(0, CH)
    def _(r):
        for k in range(8):
            rows[r, pl.ds(k * 16, 16)] = zero16

    @pl.loop(0, RPT // CH)
    def _(j):
        pltpu.sync_copy(rows, acc.at[pl.ds(sid * RPT + j * CH, CH)])

    plsc.subcore_barrier()

    @pl.loop(0, NCHUNK)
    def _(c):
        pltpu.sync_copy(src3_hbm.at[wid, c], sbuf)
        pltpu.sync_copy(dst3_hbm.at[wid, c], dbuf)
        pltpu.sync_copy(fidx3_hbm.at[wid, c], fbuf)
        pltpu.async_copy(table_hbm.at[sbuf], rows, sem).wait()
        pltpu.async_copy(feat_hbm.at[fbuf], feat_v, sem).wait()

        @pl.loop(0, CH)
        def _(r):
            for k in range(8):
                a = rows[r, pl.ds(k * 16, 16)]
                b = feat_v[r, pl.ds(k * 16, 16)]
                if is_gin:
                    rows[r, pl.ds(k * 16, 16)] = jnp.maximum(a + b, 0.0)
                else:
                    rows[r, pl.ds(k * 16, 16)] = a * b

        pltpu.sync_copy(rows, acc.at[dbuf], add=True)

    plsc.subcore_barrier()

    pltpu.sync_copy(acc.at[pl.ds(sid * RPT, RPT)],
                    p_hbm.at[cid].at[pl.ds(sid * RPT, RPT)])


def _sc_conv(table, feat, src3, dst3, fidx3, is_gin):
    p = pl.kernel(
        functools.partial(_conv_body, is_gin),
        out_type=jax.ShapeDtypeStruct((2, NACC, H), jnp.float32),
        mesh=_sc_mesh(),
        scratch_types=[
            pltpu.VMEM_SHARED((NACC, H), jnp.float32),
            pltpu.VMEM((CH, H), jnp.float32),
            pltpu.VMEM((CH, H), jnp.float32),
            pltpu.VMEM((CH,), jnp.int32),
            pltpu.VMEM((CH,), jnp.int32),
            pltpu.VMEM((CH,), jnp.int32),
            pltpu.SemaphoreType.DMA,
        ],
    )(table, feat, src3, dst3, fidx3)
    return p[0], p[1]


def _pair_body(ng_hbm, nl_hbm, src3_hbm, dst3_hbm, pg_hbm, pl_hbm,
               sidx2, didx2, rs, rd, sem):
    cid = lax.axis_index("c")
    sid = lax.axis_index("s")
    wid = sid * 2 + cid
    base = wid * EW

    pltpu.sync_copy(src3_hbm.at[wid], sidx2)
    pltpu.sync_copy(dst3_hbm.at[wid], didx2)

    @pl.loop(0, NCHUNK)
    def _(c):
        pltpu.async_copy(ng_hbm.at[sidx2.at[c]], rs, sem).wait()
        pltpu.async_copy(ng_hbm.at[didx2.at[c]], rd, sem).wait()

        @pl.loop(0, CH)
        def _(r):
            for k in range(8):
                rs[r, pl.ds(k * 16, 16)] = (rs[r, pl.ds(k * 16, 16)]
                                            * rd[r, pl.ds(k * 16, 16)])

        pltpu.sync_copy(rs, pg_hbm.at[pl.ds(base + c * CH, CH)])

        pltpu.async_copy(nl_hbm.at[sidx2.at[c]], rs, sem).wait()
        pltpu.async_copy(nl_hbm.at[didx2.at[c]], rd, sem).wait()

        @pl.loop(0, CH)
        def _(r):
            for k in range(8):
                rs[r, pl.ds(k * 16, 16)] = (rs[r, pl.ds(k * 16, 16)]
                                            * rd[r, pl.ds(k * 16, 16)])

        pltpu.sync_copy(rs, pl_hbm.at[pl.ds(base + c * CH, CH)])


def _sc_pair(node_g, node_l, src3, dst3):
    out = [jax.ShapeDtypeStruct((EP, H), jnp.float32)] * 2
    return pl.kernel(
        _pair_body,
        out_type=out,
        mesh=_sc_mesh(),
        scratch_types=[
            pltpu.VMEM((NCHUNK, CH), jnp.int32),
            pltpu.VMEM((NCHUNK, CH), jnp.int32),
            pltpu.VMEM((CH, H), jnp.float32),
            pltpu.VMEM((CH, H), jnp.float32),
            pltpu.SemaphoreType.DMA,
        ],
    )(node_g, node_l, src3, dst3)


def kernel(atom_type, edge_index, edge_type, batch_idx, pos_perturbed,
           time_step, spectrum, params):
    with jax.default_matmul_precision("default"):
        return _kernel_impl(atom_type, edge_index, edge_type, batch_idx,
                            pos_perturbed, time_step, spectrum, params)


def _kernel_impl(atom_type, edge_index, edge_type, batch_idx, pos_perturbed,
                 time_step, spectrum, params):
    P = params
    src = edge_index[0].astype(jnp.int32)
    dst = edge_index[1].astype(jnp.int32)
    pad = EP - E
    srcp = jnp.concatenate([src, jnp.zeros((pad,), jnp.int32)])
    dstp = jnp.concatenate([dst, jnp.zeros((pad,), jnp.int32)])
    etp = jnp.concatenate([edge_type.astype(jnp.int32),
                           jnp.zeros((pad,), jnp.int32)])

    ts2 = time_step.astype(jnp.float32)[:, None]
    cond = _cond(ts2, spectrum, P)
    cond_p = jnp.zeros((128, H), jnp.float32).at[:G].set(cond)
    batch2 = batch_idx.astype(jnp.int32)[:, None]

    atom2 = atom_type.astype(jnp.int32)[:, None]
    EmbG_p = jnp.zeros((128, H), jnp.float32).at[:100].set(P["EmbG"])
    EmbL_p = jnp.zeros((128, H), jnp.float32).at[:100].set(P["EmbL"])
    h, hl, x = _node_init(atom2, EmbG_p, EmbL_p, P["Wl1"][0])

    d = pos_perturbed[dst] - pos_perturbed[src]
    edge_length = jnp.sqrt(jnp.sum(d * d, axis=-1, keepdims=True) + 1e-12)
    lnp = jnp.concatenate([edge_length,
                           jnp.ones((EP - E, 1), jnp.float32)], axis=0)

    eg, el, wc0, wc1, wc2, wc3, mf, sdst, gdst = _edge_encode(
        lnp, etp[:, None], dstp[:, None], P)
    wcs = (wc0, wc1, wc2, wc3)

    src3 = srcp.reshape(NW, NCHUNK, CH)
    dst3 = dstp.reshape(NW, NCHUNK, CH)

    perm = jnp.argsort(sdst[:, 0], stable=True).astype(jnp.int32)
    ssrc3 = srcp[perm].reshape(NW, NCHUNK, CH)
    ssdst3 = sdst[:, 0][perm].reshape(NW, NCHUNK, CH)
    sgdst3 = gdst[:, 0][perm].reshape(NW, NCHUNK, CH)
    perm3 = perm.reshape(NW, NCHUNK, CH)

    def seg_sum(msg_feat, table, idx3, is_gin):
        p0, p1 = _sc_conv(table, msg_feat, ssrc3, idx3, perm3, is_gin)
        return p0[:N], p1[:N]

    for i in range(NCONV):
        p0, p1 = seg_sum(wcs[i], x, ssdst3, False)
        out = _schnet_update(p0, p1, h, P, i, batch2, cond_p)
        if i < NCONV - 1:
            h, x = out
        else:
            node_g = out[0]

    hl, el_gin, _ = lax.optimization_barrier((hl, el, node_g))
    for i in range(NCONV):
        p0, p1 = seg_sum(el_gin, hl, sgdst3, True)
        hl = _gin_update(p0, p1, hl, P, i, batch2, cond_p)[0]
    node_l = hl

    pair_g, pair_l = _sc_pair(node_g, node_l, src3, dst3)

    eig, eil = _final_edges(pair_g, eg, pair_l, el, mf, P)

    mask = mf[:E, 0] > 0.0
    return (eig[:E], eil[:E], edge_index, edge_type, edge_length, mask)
